# baseline (speedup 1.0000x reference)
import jax
import jax.numpy as jnp
from jax import lax
from jax.experimental import pallas as pl
from jax.experimental.pallas import tpu as pltpu
from jax.experimental.pallas import tpu_sc as plsc

VOCAB = 1000000
EMBED = 64
B = 16384
P = 20
N = 50
R = P + N

NUM_CORES = 2
NUM_SUBCORES = 16
NW = NUM_CORES * NUM_SUBCORES
B_PER_W = B // NW
C = 8
NCHUNK = B_PER_W // C


EH = EMBED // 2


def _sc_scores_body(c_hbm, pw_hbm, nw_hbm, w_hbm,
                    sall_hbm,
                    idx_p_all, idx_n_all, c_rows, rows3, s_all,
                    sem_g0, sem_g1, sem_o0, sem_o1):
  wid = lax.axis_index("s") * NUM_CORES + lax.axis_index("c")
  wbase = wid * B_PER_W
  lane = lax.iota(jnp.int32, 16)
  last_lane = lane == 15
  dvecs = [lane + 16 * k for k in range(EH // 16)]
  sems_g = [sem_g0, sem_g1]
  sems_o = [sem_o0, sem_o1]

  pltpu.sync_copy(pw_hbm.at[pl.ds(wbase, B_PER_W)], idx_p_all)
  pltpu.sync_copy(nw_hbm.at[pl.ds(wbase, B_PER_W)], idx_n_all)

  def gather_chunk(t, sl):
    base = t * C
    pltpu.async_copy(
        c_hbm.at[pl.ds(wbase + base, C)], c_rows.at[sl], sems_g[sl])
    for i in range(C):
      pltpu.async_copy(
          w_hbm.at[idx_p_all.at[base + i]],
          rows3.at[sl, i, pl.ds(0, P)], sems_g[sl])
      pltpu.async_copy(
          w_hbm.at[idx_n_all.at[base + i]],
          rows3.at[sl, i, pl.ds(P, N)], sems_g[sl])

  def wait_chunk(sl):
    pltpu.make_async_copy(
        c_hbm.at[pl.ds(wbase, C)], c_rows.at[sl],
        sems_g[sl]).wait()
    for i in range(C):
      pltpu.make_async_copy(
          w_hbm.at[idx_p_all.at[i]],
          rows3.at[sl, i, pl.ds(0, P)], sems_g[sl]).wait()
      pltpu.make_async_copy(
          w_hbm.at[idx_n_all.at[i]],
          rows3.at[sl, i, pl.ds(P, N)], sems_g[sl]).wait()

  def compute_chunk(sl):
    b_vec = jnp.full((16,), sl, jnp.int32)

    G = 5

    def center_body(i, carry2):
      i_vec = jnp.full((16,), i, jnp.int32)
      cvec = [c_rows[sl, i, pl.ds(16 * k, 16)] for k in range(2)]
      for j0 in range(0, R, G):
        loads = [[rows3[sl, i, j0 + g, pl.ds(16 * k, 16)] for k in range(2)]
                 for g in range(G)]
        accs = [l[0] * cvec[0] + l[1] * cvec[1] for l in loads]
        cums = [plsc.cumsum(a) for a in accs]
        for g, cum in enumerate(cums):
          j_vec = jnp.full((16,), j0 + g, jnp.int32)
          plsc.store_scatter(s_all, [b_vec, i_vec, j_vec], cum, mask=last_lane)
      return carry2

    lax.fori_loop(0, C, center_body, 0, unroll=False)

  def out_copy(t, sl):
    pltpu.async_copy(
        s_all.at[sl], sall_hbm.at[pl.ds(wbase + t * C, C)], sems_o[sl])

  def wait_out(sl):
    pltpu.make_async_copy(
        s_all.at[sl], sall_hbm.at[pl.ds(wbase, C)], sems_o[sl]).wait()

  gather_chunk(0, 0)

  def outer(tt, carry):
    for b in range(2):
      t = tt * 2 + b

      @pl.when(t + 1 < NCHUNK)
      def _():
        gather_chunk(t + 1, 1 - b)

      wait_chunk(b)

      @pl.when(t >= 2)
      def _():
        wait_out(b)

      compute_chunk(b)
      out_copy(t, b)
    return carry

  lax.fori_loop(0, NCHUNK // 2, outer, 0, unroll=False)
  wait_out(0)
  wait_out(1)


@jax.jit
def _sc_scores(c, pw, nw, W_out):
  mesh = plsc.VectorSubcoreMesh(
      core_axis_name="c", subcore_axis_name="s",
      num_cores=NUM_CORES, num_subcores=NUM_SUBCORES)
  k = pl.kernel(
      _sc_scores_body,
      out_type=jax.ShapeDtypeStruct((B, R), jnp.float32),
      mesh=mesh,
      compiler_params=pltpu.CompilerParams(
          needs_layout_passes=False, use_tc_tiling_on_sc=False),
      scratch_types=[
          pltpu.VMEM((B_PER_W, P), jnp.int32),
          pltpu.VMEM((B_PER_W, N), jnp.int32),
          pltpu.VMEM((2, C, EH), jnp.float32),
          pltpu.VMEM((2, C, R, EH), jnp.float32),
          pltpu.VMEM((2, C, R), jnp.float32),
          pltpu.SemaphoreType.DMA,
          pltpu.SemaphoreType.DMA,
          pltpu.SemaphoreType.DMA,
          pltpu.SemaphoreType.DMA,
      ],
  )
  return k(c, pw, nw, W_out)


def _loss_body(sl_ref, sr_ref, out_ref):
  s = sl_ref[...] + sr_ref[...]

  def logsig(x):
    return jnp.minimum(x, 0.0) - jnp.log1p(jnp.exp(-jnp.abs(x)))

  per_b = jnp.sum(logsig(s[:, :P]), axis=1) + jnp.sum(logsig(-s[:, P:]), axis=1)
  out_ref[0, 0] = -jnp.sum(per_b) / B


@jax.jit
def _tc_loss(s_l, s_r):
  out = pl.pallas_call(
      _loss_body,
      out_shape=jax.ShapeDtypeStruct((1, 1), jnp.float32),
      out_specs=pl.BlockSpec(memory_space=pltpu.SMEM),
  )(s_l, s_r)
  return out[0, 0]


def kernel(centerWords, positiveWords, negativeWords, W_in, W_out):
  return _run(centerWords, positiveWords, negativeWords, W_in, W_out)


EH = EMBED // 2


@jax.jit
def _run(centerWords, positiveWords, negativeWords, W_in, W_out):
  cw = centerWords.astype(jnp.int32)
  pw = positiveWords.astype(jnp.int32)
  nw = negativeWords.astype(jnp.int32)
  c = jnp.take_along_axis(
      W_in, jnp.broadcast_to(cw[:, None], (B, EMBED)), axis=0)
  s_l = _sc_scores(c[:, :EH], pw, nw, W_out[:, :EH])
  s_r = _sc_scores(c[:, EH:], pw, nw, W_out[:, EH:])
  return _tc_loss(s_l, s_r)


# speedup vs baseline: 1.8525x; 1.8525x over previous
import jax
import jax.numpy as jnp
from jax import lax
from jax.experimental import pallas as pl
from jax.experimental.pallas import tpu as pltpu
from jax.experimental.pallas import tpu_sc as plsc

VOCAB = 1000000
EMBED = 64
B = 16384
P = 20
N = 50
R = P + N

NUM_CORES = 2
NUM_SUBCORES = 16
NW = NUM_CORES * NUM_SUBCORES
B_PER_W = B // NW
C = 8
NCHUNK = B_PER_W // C


def _sc_scores_body(c_hbm, pw_hbm, nw_hbm, w_hbm,
                    sall_hbm,
                    idx_p_all, idx_n_all, c_rows, rows3, s_all,
                    sem_g0, sem_g1, sem_o0, sem_o1):
  wid = lax.axis_index("s") * NUM_CORES + lax.axis_index("c")
  wbase = wid * B_PER_W
  lane = lax.iota(jnp.int32, 16)
  last_lane = lane == 15
  dvecs = [lane + 16 * k for k in range(4)]
  sems_g = [sem_g0, sem_g1]
  sems_o = [sem_o0, sem_o1]

  pltpu.sync_copy(pw_hbm.at[pl.ds(wbase, B_PER_W)], idx_p_all)
  pltpu.sync_copy(nw_hbm.at[pl.ds(wbase, B_PER_W)], idx_n_all)

  def gather_chunk(t, sl):
    base = t * C
    pltpu.async_copy(
        c_hbm.at[pl.ds(wbase + base, C)], c_rows.at[sl], sems_g[sl])
    for i in range(C):
      pltpu.async_copy(
          w_hbm.at[idx_p_all.at[base + i]],
          rows3.at[sl, i, pl.ds(0, P)], sems_g[sl])
      pltpu.async_copy(
          w_hbm.at[idx_n_all.at[base + i]],
          rows3.at[sl, i, pl.ds(P, N)], sems_g[sl])

  def wait_chunk(sl):
    pltpu.make_async_copy(
        c_hbm.at[pl.ds(wbase, C)], c_rows.at[sl],
        sems_g[sl]).wait()
    for i in range(C):
      pltpu.make_async_copy(
          w_hbm.at[idx_p_all.at[i]],
          rows3.at[sl, i, pl.ds(0, P)], sems_g[sl]).wait()
      pltpu.make_async_copy(
          w_hbm.at[idx_n_all.at[i]],
          rows3.at[sl, i, pl.ds(P, N)], sems_g[sl]).wait()

  def compute_chunk(sl):
    b_vec = jnp.full((16,), sl, jnp.int32)

    G = 5

    def center_body(i, carry2):
      i_vec = jnp.full((16,), i, jnp.int32)
      cvec = [c_rows[sl, i, pl.ds(16 * k, 16)] for k in range(4)]
      for j0 in range(0, R, G):
        loads = [[rows3[sl, i, j0 + g, pl.ds(16 * k, 16)] for k in range(4)]
                 for g in range(G)]
        accs = [(l[0] * cvec[0] + l[1] * cvec[1])
                + (l[2] * cvec[2] + l[3] * cvec[3]) for l in loads]
        cums = [plsc.cumsum(a) for a in accs]
        for g, cum in enumerate(cums):
          j_vec = jnp.full((16,), j0 + g, jnp.int32)
          plsc.store_scatter(s_all, [b_vec, i_vec, j_vec], cum, mask=last_lane)
      return carry2

    lax.fori_loop(0, C, center_body, 0, unroll=False)

  def out_copy(t, sl):
    pltpu.async_copy(
        s_all.at[sl], sall_hbm.at[pl.ds(wbase + t * C, C)], sems_o[sl])

  def wait_out(sl):
    pltpu.make_async_copy(
        s_all.at[sl], sall_hbm.at[pl.ds(wbase, C)], sems_o[sl]).wait()

  gather_chunk(0, 0)

  def outer(tt, carry):
    for b in range(2):
      t = tt * 2 + b

      @pl.when(t + 1 < NCHUNK)
      def _():
        gather_chunk(t + 1, 1 - b)

      wait_chunk(b)

      @pl.when(t >= 2)
      def _():
        wait_out(b)

      compute_chunk(b)
      out_copy(t, b)
    return carry

  lax.fori_loop(0, NCHUNK // 2, outer, 0, unroll=False)
  wait_out(0)
  wait_out(1)


@jax.jit
def _sc_scores(c, pw, nw, W_out):
  mesh = plsc.VectorSubcoreMesh(
      core_axis_name="c", subcore_axis_name="s",
      num_cores=NUM_CORES, num_subcores=NUM_SUBCORES)
  k = pl.kernel(
      _sc_scores_body,
      out_type=jax.ShapeDtypeStruct((B, R), jnp.float32),
      mesh=mesh,
      compiler_params=pltpu.CompilerParams(
          needs_layout_passes=False, use_tc_tiling_on_sc=False),
      scratch_types=[
          pltpu.VMEM((B_PER_W, P), jnp.int32),
          pltpu.VMEM((B_PER_W, N), jnp.int32),
          pltpu.VMEM((2, C, EMBED), jnp.float32),
          pltpu.VMEM((2, C, R, EMBED), jnp.float32),
          pltpu.VMEM((2, C, R), jnp.float32),
          pltpu.SemaphoreType.DMA,
          pltpu.SemaphoreType.DMA,
          pltpu.SemaphoreType.DMA,
          pltpu.SemaphoreType.DMA,
      ],
  )
  return k(c, pw, nw, W_out)


def _loss_body(s_ref, out_ref):
  s = s_ref[...]

  def logsig(x):
    return jnp.minimum(x, 0.0) - jnp.log1p(jnp.exp(-jnp.abs(x)))

  per_b = jnp.sum(logsig(s[:, :P]), axis=1) + jnp.sum(logsig(-s[:, P:]), axis=1)
  out_ref[0, 0] = -jnp.sum(per_b) / B


@jax.jit
def _tc_loss(s_all):
  out = pl.pallas_call(
      _loss_body,
      out_shape=jax.ShapeDtypeStruct((1, 1), jnp.float32),
      out_specs=pl.BlockSpec(memory_space=pltpu.SMEM),
  )(s_all)
  return out[0, 0]


def kernel(centerWords, positiveWords, negativeWords, W_in, W_out):
  return _run(centerWords, positiveWords, negativeWords, W_in, W_out)


@jax.jit
def _run(centerWords, positiveWords, negativeWords, W_in, W_out):
  cw = centerWords.astype(jnp.int32)
  pw = positiveWords.astype(jnp.int32)
  nw = negativeWords.astype(jnp.int32)
  c = jnp.take_along_axis(
      W_in, jnp.broadcast_to(cw[:, None], (B, EMBED)), axis=0)
  s_all = _sc_scores(c, pw, nw, W_out)
  return _tc_loss(s_all)


# speedup vs baseline: 1.8559x; 1.0018x over previous
import jax
import jax.numpy as jnp
from jax import lax
from jax.experimental import pallas as pl
from jax.experimental.pallas import tpu as pltpu
from jax.experimental.pallas import tpu_sc as plsc

VOCAB = 1000000
EMBED = 64
B = 16384
P = 20
N = 50
R = P + N

NUM_CORES = 2
NUM_SUBCORES = 16
NW = NUM_CORES * NUM_SUBCORES
B_PER_W = B // NW
C = 8
NCHUNK = B_PER_W // C


def _sc_scores_body(c_hbm, pw_hbm, nw_hbm, w_hbm,
                    sall_hbm,
                    idx_p_all, idx_n_all, c_rows, rows3, s_all,
                    sem_g0, sem_g1, sem_o0, sem_o1):
  wid = lax.axis_index("s") * NUM_CORES + lax.axis_index("c")
  wbase = wid * B_PER_W
  lane = lax.iota(jnp.int32, 16)
  last_lane = lane == 15
  sems_g = [sem_g0, sem_g1]
  sems_o = [sem_o0, sem_o1]

  pltpu.sync_copy(pw_hbm.at[pl.ds(wbase, B_PER_W)], idx_p_all)
  pltpu.sync_copy(nw_hbm.at[pl.ds(wbase, B_PER_W)], idx_n_all)

  def gather_chunk(t, sl):
    base = t * C
    pltpu.async_copy(
        c_hbm.at[pl.ds(wbase + base, C)], c_rows.at[sl], sems_g[sl])
    for i in range(C):
      pltpu.async_copy(
          w_hbm.at[idx_p_all.at[base + i]],
          rows3.at[sl, i, pl.ds(0, P)], sems_g[sl])
      pltpu.async_copy(
          w_hbm.at[idx_n_all.at[base + i]],
          rows3.at[sl, i, pl.ds(P, N)], sems_g[sl])

  def wait_chunk(sl):
    pltpu.make_async_copy(
        c_hbm.at[pl.ds(wbase, C)], c_rows.at[sl],
        sems_g[sl]).wait()
    for i in range(C):
      pltpu.make_async_copy(
          w_hbm.at[idx_p_all.at[i]],
          rows3.at[sl, i, pl.ds(0, P)], sems_g[sl]).wait()
      pltpu.make_async_copy(
          w_hbm.at[idx_n_all.at[i]],
          rows3.at[sl, i, pl.ds(P, N)], sems_g[sl]).wait()

  def compute_chunk(sl):
    b_vec = jnp.full((16,), sl, jnp.int32)

    G = 5

    def center_body(i, carry2):
      i_vec = jnp.full((16,), i, jnp.int32)
      cvec = [c_rows[sl, i, pl.ds(16 * k, 16)] for k in range(4)]
      for j0 in range(0, R, G):
        loads = [[rows3[sl, i, j0 + g, pl.ds(16 * k, 16)] for k in range(4)]
                 for g in range(G)]
        accs = [(l[0] * cvec[0] + l[1] * cvec[1])
                + (l[2] * cvec[2] + l[3] * cvec[3]) for l in loads]
        cums = [plsc.cumsum(a) for a in accs]
        for g, cum in enumerate(cums):
          j_vec = jnp.full((16,), j0 + g, jnp.int32)
          plsc.store_scatter(s_all, [b_vec, i_vec, j_vec], cum, mask=last_lane)
      return carry2

    lax.fori_loop(0, C, center_body, 0, unroll=False)

  def out_copy(t, sl):
    pltpu.async_copy(
        s_all.at[sl], sall_hbm.at[pl.ds(wbase + t * C, C)], sems_o[sl])

  def wait_out(sl):
    pltpu.make_async_copy(
        s_all.at[sl], sall_hbm.at[pl.ds(wbase, C)], sems_o[sl]).wait()

  gather_chunk(0, 0)

  def outer(tt, carry):
    for b in range(2):
      t = tt * 2 + b

      @pl.when(t + 1 < NCHUNK)
      def _():
        gather_chunk(t + 1, 1 - b)

      wait_chunk(b)

      @pl.when(t >= 2)
      def _():
        wait_out(b)

      compute_chunk(b)
      out_copy(t, b)
    return carry

  lax.fori_loop(0, NCHUNK // 2, outer, 0, unroll=False)
  wait_out(0)
  wait_out(1)


@jax.jit
def _sc_scores(c, pw, nw, W_out):
  mesh = plsc.VectorSubcoreMesh(
      core_axis_name="c", subcore_axis_name="s",
      num_cores=NUM_CORES, num_subcores=NUM_SUBCORES)
  k = pl.kernel(
      _sc_scores_body,
      out_type=jax.ShapeDtypeStruct((B, R), jnp.float32),
      mesh=mesh,
      compiler_params=pltpu.CompilerParams(
          needs_layout_passes=False, use_tc_tiling_on_sc=False),
      scratch_types=[
          pltpu.VMEM((B_PER_W, P), jnp.int32),
          pltpu.VMEM((B_PER_W, N), jnp.int32),
          pltpu.VMEM((2, C, EMBED), jnp.float32),
          pltpu.VMEM((2, C, R, EMBED), jnp.float32),
          pltpu.VMEM((2, C, R), jnp.float32),
          pltpu.SemaphoreType.DMA,
          pltpu.SemaphoreType.DMA,
          pltpu.SemaphoreType.DMA,
          pltpu.SemaphoreType.DMA,
      ],
  )
  return k(c, pw, nw, W_out)


def _loss_body(s_ref, out_ref):
  s = s_ref[...]

  def logsig(x):
    return jnp.minimum(x, 0.0) - jnp.log1p(jnp.exp(-jnp.abs(x)))

  per_b = jnp.sum(logsig(s[:, :P]), axis=1) + jnp.sum(logsig(-s[:, P:]), axis=1)
  out_ref[0, 0] = -jnp.sum(per_b) / B


@jax.jit
def _tc_loss(s_all):
  out = pl.pallas_call(
      _loss_body,
      out_shape=jax.ShapeDtypeStruct((1, 1), jnp.float32),
      out_specs=pl.BlockSpec(memory_space=pltpu.SMEM),
  )(s_all)
  return out[0, 0]


def kernel(centerWords, positiveWords, negativeWords, W_in, W_out):
  return _run(centerWords, positiveWords, negativeWords, W_in, W_out)


@jax.jit
def _run(centerWords, positiveWords, negativeWords, W_in, W_out):
  cw = centerWords.astype(jnp.int32)
  pw = positiveWords.astype(jnp.int32)
  nw = negativeWords.astype(jnp.int32)
  c = jnp.take_along_axis(
      W_in, jnp.broadcast_to(cw[:, None], (B, EMBED)), axis=0)
  s_all = _sc_scores(c, pw, nw, W_out)
  return _tc_loss(s_all)


# speedup vs baseline: 1.8600x; 1.0022x over previous
import jax
import jax.numpy as jnp
from jax import lax
from jax.experimental import pallas as pl
from jax.experimental.pallas import tpu as pltpu
from jax.experimental.pallas import tpu_sc as plsc

VOCAB = 1000000
EMBED = 64
B = 16384
P = 20
N = 50
R = P + N

NUM_CORES = 2
NUM_SUBCORES = 16
NW = NUM_CORES * NUM_SUBCORES
B_PER_W = B // NW
C = 8
NCHUNK = B_PER_W // C


def _sc_scores_body(c_hbm, pw_hbm, nw_hbm, w_hbm,
                    sall_hbm,
                    idx_p_all, idx_n_all, c_rows, rows3, s_all,
                    sem_g0, sem_g1, sem_o0, sem_o1):
  wid = lax.axis_index("s") * NUM_CORES + lax.axis_index("c")
  wbase = wid * B_PER_W
  lane = lax.iota(jnp.int32, 16)
  last_lane = lane == 15
  sems_g = [sem_g0, sem_g1]
  sems_o = [sem_o0, sem_o1]

  pltpu.sync_copy(pw_hbm.at[pl.ds(wbase, B_PER_W)], idx_p_all)
  pltpu.sync_copy(nw_hbm.at[pl.ds(wbase, B_PER_W)], idx_n_all)

  def gather_chunk(t, sl):
    base = t * C
    pltpu.async_copy(
        c_hbm.at[pl.ds(wbase + base, C)], c_rows.at[sl], sems_g[sl])
    for i in range(C):
      pltpu.async_copy(
          w_hbm.at[idx_p_all.at[base + i]],
          rows3.at[sl, i, pl.ds(0, P)], sems_g[sl])
      pltpu.async_copy(
          w_hbm.at[idx_n_all.at[base + i]],
          rows3.at[sl, i, pl.ds(P, N)], sems_g[sl])

  def wait_chunk(sl):
    pltpu.make_async_copy(
        c_hbm.at[pl.ds(wbase, C)], c_rows.at[sl],
        sems_g[sl]).wait()
    pltpu.make_async_copy(
        w_hbm.at[idx_p_all.at[0]], rows3.at[sl], sems_g[sl]).wait()

  def compute_chunk(sl):
    b_vec = jnp.full((16,), sl, jnp.int32)

    G = 5

    def center_body(i, carry2):
      i_vec = jnp.full((16,), i, jnp.int32)
      cvec = [c_rows[sl, i, pl.ds(16 * k, 16)] for k in range(4)]
      for j0 in range(0, R, G):
        loads = [[rows3[sl, i, j0 + g, pl.ds(16 * k, 16)] for k in range(4)]
                 for g in range(G)]
        accs = [(l[0] * cvec[0] + l[1] * cvec[1])
                + (l[2] * cvec[2] + l[3] * cvec[3]) for l in loads]
        cums = [plsc.cumsum(a) for a in accs]
        for g, cum in enumerate(cums):
          j_vec = jnp.full((16,), j0 + g, jnp.int32)
          plsc.store_scatter(s_all, [b_vec, i_vec, j_vec], cum, mask=last_lane)
      return carry2

    lax.fori_loop(0, C, center_body, 0, unroll=False)

  def out_copy(t, sl):
    pltpu.async_copy(
        s_all.at[sl], sall_hbm.at[pl.ds(wbase + t * C, C)], sems_o[sl])

  def wait_out(sl):
    pltpu.make_async_copy(
        s_all.at[sl], sall_hbm.at[pl.ds(wbase, C)], sems_o[sl]).wait()

  gather_chunk(0, 0)

  def outer(tt, carry):
    for b in range(2):
      t = tt * 2 + b

      @pl.when(t + 1 < NCHUNK)
      def _():
        gather_chunk(t + 1, 1 - b)

      wait_chunk(b)

      @pl.when(t >= 2)
      def _():
        wait_out(b)

      compute_chunk(b)
      out_copy(t, b)
    return carry

  lax.fori_loop(0, NCHUNK // 2, outer, 0, unroll=False)
  wait_out(0)
  wait_out(1)


@jax.jit
def _sc_scores(c, pw, nw, W_out):
  mesh = plsc.VectorSubcoreMesh(
      core_axis_name="c", subcore_axis_name="s",
      num_cores=NUM_CORES, num_subcores=NUM_SUBCORES)
  k = pl.kernel(
      _sc_scores_body,
      out_type=jax.ShapeDtypeStruct((B, R), jnp.float32),
      mesh=mesh,
      compiler_params=pltpu.CompilerParams(
          needs_layout_passes=False, use_tc_tiling_on_sc=False),
      scratch_types=[
          pltpu.VMEM((B_PER_W, P), jnp.int32),
          pltpu.VMEM((B_PER_W, N), jnp.int32),
          pltpu.VMEM((2, C, EMBED), jnp.float32),
          pltpu.VMEM((2, C, R, EMBED), jnp.float32),
          pltpu.VMEM((2, C, R), jnp.float32),
          pltpu.SemaphoreType.DMA,
          pltpu.SemaphoreType.DMA,
          pltpu.SemaphoreType.DMA,
          pltpu.SemaphoreType.DMA,
      ],
  )
  return k(c, pw, nw, W_out)


def _loss_body(s_ref, out_ref):
  s = s_ref[...]

  def logsig(x):
    return jnp.minimum(x, 0.0) - jnp.log1p(jnp.exp(-jnp.abs(x)))

  per_b = jnp.sum(logsig(s[:, :P]), axis=1) + jnp.sum(logsig(-s[:, P:]), axis=1)
  out_ref[0, 0] = -jnp.sum(per_b) / B


@jax.jit
def _tc_loss(s_all):
  out = pl.pallas_call(
      _loss_body,
      out_shape=jax.ShapeDtypeStruct((1, 1), jnp.float32),
      out_specs=pl.BlockSpec(memory_space=pltpu.SMEM),
  )(s_all)
  return out[0, 0]


def kernel(centerWords, positiveWords, negativeWords, W_in, W_out):
  return _run(centerWords, positiveWords, negativeWords, W_in, W_out)


@jax.jit
def _run(centerWords, positiveWords, negativeWords, W_in, W_out):
  cw = centerWords.astype(jnp.int32)
  pw = positiveWords.astype(jnp.int32)
  nw = negativeWords.astype(jnp.int32)
  c = jnp.take_along_axis(
      W_in, jnp.broadcast_to(cw[:, None], (B, EMBED)), axis=0)
  s_all = _sc_scores(c, pw, nw, W_out)
  return _tc_loss(s_all)
